# baseline (speedup 1.0000x reference)
import functools

import jax
import jax.numpy as jnp
from jax import lax
from jax.experimental import pallas as pl
from jax.experimental.pallas import tpu as pltpu
from jax.experimental.pallas import tpu_sc as plsc

N_NODES = 10000
N_PAD = 10240
N_EDGES = 160000
D = 256
H = D // 2
CHUNK = 128
N_TILES = 16
K_PER_TILE = 80
N_CHUNKS = N_TILES * K_PER_TILE
E_PADDED = N_CHUNKS * CHUNK
ROWS_PER_TILE = N_PAD // N_TILES


def _sc_segment_sum(xh, src_idx, dst_idx, zeros):
    mesh = plsc.VectorSubcoreMesh(core_axis_name="c", subcore_axis_name="s")

    @functools.partial(
        pl.kernel,
        out_type=jax.ShapeDtypeStruct((2 * N_PAD, H), jnp.float32),
        mesh=mesh,
        scratch_types=[
            pltpu.VMEM((K_PER_TILE, CHUNK), jnp.int32),
            pltpu.VMEM((K_PER_TILE, CHUNK), jnp.int32),
            pltpu.VMEM((CHUNK, H), jnp.float32),
            pltpu.VMEM_SHARED((N_PAD, H), jnp.float32),
        ],
    )
    def sc_kernel(xh_hbm, src_hbm, dst_hbm, zeros_hbm, out_hbm,
                  src_v, dst_v, rows_v, acc_sh):
        c = lax.axis_index("c")
        s = lax.axis_index("s")
        base = s * K_PER_TILE

        pltpu.sync_copy(src_hbm.at[c, pl.ds(base, K_PER_TILE)], src_v)
        pltpu.sync_copy(dst_hbm.at[pl.ds(base, K_PER_TILE)], dst_v)
        pltpu.sync_copy(zeros_hbm, acc_sh.at[pl.ds(s * ROWS_PER_TILE, ROWS_PER_TILE)])
        plsc.subcore_barrier()

        @pl.loop(0, K_PER_TILE)
        def _(k):
            pltpu.sync_copy(xh_hbm.at[src_v.at[k]], rows_v)
            pltpu.sync_copy(rows_v, acc_sh.at[dst_v.at[k]], add=True)

        plsc.subcore_barrier()
        pltpu.sync_copy(
            acc_sh.at[pl.ds(s * ROWS_PER_TILE, ROWS_PER_TILE)],
            out_hbm.at[pl.ds(c * N_PAD + s * ROWS_PER_TILE, ROWS_PER_TILE)],
        )

    return sc_kernel(xh, src_idx, dst_idx, zeros)


def _tc_finish_body(a0_ref, a1_ref, x_ref, w0_ref, w1_ref, wr_ref, b_ref, o_ref):
    y = jnp.dot(a0_ref[...], w0_ref[...],
                preferred_element_type=jnp.float32,
                precision=jax.lax.Precision.HIGHEST)
    y = y + jnp.dot(a1_ref[...], w1_ref[...],
                    preferred_element_type=jnp.float32,
                    precision=jax.lax.Precision.HIGHEST)
    y = y + jnp.dot(x_ref[...], wr_ref[...],
                    preferred_element_type=jnp.float32,
                    precision=jax.lax.Precision.HIGHEST)
    y = y + b_ref[...]
    m = jnp.max(y, axis=-1, keepdims=True)
    t = y - m
    lse = jnp.log(jnp.sum(jnp.exp(t), axis=-1, keepdims=True))
    o_ref[...] = t - lse


def _tc_finish(agg0, agg1, x, w0, w1, wr, b2d):
    n = x.shape[0]
    blk = 1000
    return pl.pallas_call(
        _tc_finish_body,
        grid=(n // blk,),
        in_specs=[
            pl.BlockSpec((blk, H), lambda i: (i, 0)),
            pl.BlockSpec((blk, H), lambda i: (i, 0)),
            pl.BlockSpec((blk, D), lambda i: (i, 0)),
            pl.BlockSpec((H, D), lambda i: (0, 0)),
            pl.BlockSpec((H, D), lambda i: (0, 0)),
            pl.BlockSpec((D, D), lambda i: (0, 0)),
            pl.BlockSpec((1, D), lambda i: (0, 0)),
        ],
        out_specs=pl.BlockSpec((blk, D), lambda i: (i, 0)),
        out_shape=jax.ShapeDtypeStruct((n, D), jnp.float32),
    )(agg0, agg1, x, w0, w1, wr, b2d)


def kernel(x, edge_index, W_rel, W_root, b):
    src = edge_index[0]
    dst = edge_index[1]
    n_extra = E_PADDED - N_EDGES
    src_pad = jnp.concatenate([src, jnp.zeros((n_extra,), jnp.int32)])
    dst_pad = jnp.concatenate(
        [dst, N_NODES + (jnp.arange(n_extra, dtype=jnp.int32) % (N_PAD - N_NODES))])
    xh = jnp.concatenate([x[:, :H], x[:, H:]], axis=0)
    srcs = src_pad.reshape(N_CHUNKS, CHUNK)
    src_idx = jnp.stack([srcs, srcs + N_NODES])
    dst_idx = dst_pad.reshape(N_CHUNKS, CHUNK)
    zeros = jnp.zeros((ROWS_PER_TILE, H), jnp.float32)

    agg_cat = _sc_segment_sum(xh, src_idx, dst_idx, zeros)

    out = _tc_finish(
        agg_cat[:N_NODES], agg_cat[N_PAD:N_PAD + N_NODES], x,
        W_rel[:, :H].T, W_rel[:, H:].T, W_root.T, b.reshape(1, D),
    )
    return out


# speedup vs baseline: 1.2057x; 1.2057x over previous
import functools

import jax
import jax.numpy as jnp
from jax import lax
from jax.experimental import pallas as pl
from jax.experimental.pallas import tpu as pltpu
from jax.experimental.pallas import tpu_sc as plsc

N_NODES = 10000
N_PAD = 10240
N_EDGES = 160000
D = 256
H = D // 2
CHUNK = 128
N_TILES = 16
K_PER_TILE = 80
N_CHUNKS = N_TILES * K_PER_TILE
E_PADDED = N_CHUNKS * CHUNK
ROWS_PER_TILE = N_PAD // N_TILES
SUPER = 2


def _sc_segment_sum(xh, src_idx, dst_idx, zeros):
    mesh = plsc.VectorSubcoreMesh(core_axis_name="c", subcore_axis_name="s")

    @functools.partial(
        pl.kernel,
        out_type=jax.ShapeDtypeStruct((2 * N_PAD, H), jnp.float32),
        mesh=mesh,
        scratch_types=[
            pltpu.VMEM((CHUNK,), jnp.int32),
            pltpu.VMEM((CHUNK,), jnp.int32),
            pltpu.VMEM((CHUNK,), jnp.int32),
            pltpu.VMEM((CHUNK,), jnp.int32),
            pltpu.VMEM((CHUNK, H), jnp.float32),
            pltpu.VMEM((CHUNK, H), jnp.float32),
            pltpu.VMEM_SHARED((N_PAD, H), jnp.float32),
        ]
        + [pltpu.SemaphoreType.DMA] * 4,
    )
    def sc_kernel(xh_hbm, src_hbm, dst_hbm, zeros_hbm, out_hbm,
                  src0, src1, dst0, dst1, rows0, rows1, acc_sh,
                  sg0, sg1, ss0, ss1):
        c = lax.axis_index("c")
        s = lax.axis_index("s")
        slots = ((src0, dst0, rows0, sg0, ss0), (src1, dst1, rows1, sg1, ss1))

        def idx_load(i, sl):
            src_v, dst_v = slots[sl][0], slots[sl][1]
            pltpu.sync_copy(src_hbm.at[c, i], src_v)
            pltpu.sync_copy(dst_hbm.at[i], dst_v)

        def g_start(sl):
            src_v, rows_v, sem = slots[sl][0], slots[sl][2], slots[sl][3]
            pltpu.async_copy(xh_hbm.at[src_v], rows_v, sem)

        def g_wait(sl):
            src_v, rows_v, sem = slots[sl][0], slots[sl][2], slots[sl][3]
            pltpu.make_async_copy(xh_hbm.at[src_v], rows_v, sem).wait()

        def s_start(sl):
            dst_v, rows_v, sem = slots[sl][1], slots[sl][2], slots[sl][4]
            pltpu.async_copy(rows_v, acc_sh.at[dst_v], sem, add=True)

        def s_wait(sl):
            dst_v, rows_v, sem = slots[sl][1], slots[sl][2], slots[sl][4]
            pltpu.make_async_copy(rows_v, acc_sh.at[dst_v], sem).wait()

        idx_load(s, 0)
        g_start(0)
        pltpu.sync_copy(zeros_hbm, acc_sh.at[pl.ds(s * ROWS_PER_TILE, ROWS_PER_TILE)])
        plsc.subcore_barrier()

        idx_load(N_TILES + s, 1)
        g_wait(0)
        g_start(1)
        s_start(0)

        @pl.loop(0, (K_PER_TILE - 2) // 2)
        def _(g):
            for half in range(2):
                k = 2 * g + 1 + half
                cur, nxt = (1, 0) if half == 0 else (0, 1)
                s_wait(nxt)
                idx_load((k + 1) * N_TILES + s, nxt)
                g_wait(cur)
                g_start(nxt)
                s_start(cur)

        s_wait(0)
        g_wait(1)
        s_start(1)
        s_wait(1)

        plsc.subcore_barrier()
        pltpu.sync_copy(
            acc_sh.at[pl.ds(s * ROWS_PER_TILE, ROWS_PER_TILE)],
            out_hbm.at[pl.ds(c * N_PAD + s * ROWS_PER_TILE, ROWS_PER_TILE)],
        )

    return sc_kernel(xh, src_idx, dst_idx, zeros)


def _tc_finish_body(a0_ref, a1_ref, x_ref, w0_ref, w1_ref, wr_ref, b_ref, o_ref):
    y = jnp.dot(a0_ref[...], w0_ref[...],
                preferred_element_type=jnp.float32,
                precision=jax.lax.Precision.HIGHEST)
    y = y + jnp.dot(a1_ref[...], w1_ref[...],
                    preferred_element_type=jnp.float32,
                    precision=jax.lax.Precision.HIGHEST)
    y = y + jnp.dot(x_ref[...], wr_ref[...],
                    preferred_element_type=jnp.float32,
                    precision=jax.lax.Precision.HIGHEST)
    y = y + b_ref[...]
    m = jnp.max(y, axis=-1, keepdims=True)
    t = y - m
    lse = jnp.log(jnp.sum(jnp.exp(t), axis=-1, keepdims=True))
    o_ref[...] = t - lse


def _tc_finish(agg0, agg1, x, w0, w1, wr, b2d):
    n = x.shape[0]
    blk = 1000
    return pl.pallas_call(
        _tc_finish_body,
        grid=(n // blk,),
        in_specs=[
            pl.BlockSpec((blk, H), lambda i: (i, 0)),
            pl.BlockSpec((blk, H), lambda i: (i, 0)),
            pl.BlockSpec((blk, D), lambda i: (i, 0)),
            pl.BlockSpec((H, D), lambda i: (0, 0)),
            pl.BlockSpec((H, D), lambda i: (0, 0)),
            pl.BlockSpec((D, D), lambda i: (0, 0)),
            pl.BlockSpec((1, D), lambda i: (0, 0)),
        ],
        out_specs=pl.BlockSpec((blk, D), lambda i: (i, 0)),
        out_shape=jax.ShapeDtypeStruct((n, D), jnp.float32),
    )(agg0, agg1, x, w0, w1, wr, b2d)


def kernel(x, edge_index, W_rel, W_root, b):
    src = edge_index[0]
    dst = edge_index[1]
    n_extra = E_PADDED - N_EDGES
    src_pad = jnp.concatenate([src, jnp.zeros((n_extra,), jnp.int32)])
    dst_pad = jnp.concatenate(
        [dst, N_NODES + (jnp.arange(n_extra, dtype=jnp.int32) % (N_PAD - N_NODES))])
    xh = jnp.concatenate([x[:, :H], x[:, H:]], axis=0)
    srcs = src_pad.reshape(N_CHUNKS, CHUNK)
    src_idx = jnp.stack([srcs, srcs + N_NODES])
    dst_idx = dst_pad.reshape(N_CHUNKS, CHUNK)
    zeros = jnp.zeros((ROWS_PER_TILE, H), jnp.float32)

    agg_cat = _sc_segment_sum(xh, src_idx, dst_idx, zeros)

    out = _tc_finish(
        agg_cat[:N_NODES], agg_cat[N_PAD:N_PAD + N_NODES], x,
        W_rel[:, :H].T, W_rel[:, H:].T, W_root.T, b.reshape(1, D),
    )
    return out


# speedup vs baseline: 1.4402x; 1.1945x over previous
import functools

import jax
import jax.numpy as jnp
from jax import lax
from jax.experimental import pallas as pl
from jax.experimental.pallas import tpu as pltpu
from jax.experimental.pallas import tpu_sc as plsc

N_NODES = 10000
N_PAD = 10240
N_EDGES = 160000
D = 256
H = D // 2
CHUNK = 128
N_CHUNKS = N_EDGES // CHUNK
N_TILES = 16
ROWS_PER_TILE = N_PAD // N_TILES
MAX_CHUNKS_PER_TILE = (N_CHUNKS + N_TILES - 1) // N_TILES


def _sc_segment_sum(xh, src_idx, dst_idx, zeros):
    mesh = plsc.VectorSubcoreMesh(core_axis_name="c", subcore_axis_name="s")

    @functools.partial(
        pl.kernel,
        out_type=jax.ShapeDtypeStruct((2 * N_PAD, H), jnp.float32),
        mesh=mesh,
        scratch_types=[
            pltpu.VMEM((CHUNK,), jnp.int32),
            pltpu.VMEM((CHUNK,), jnp.int32),
            pltpu.VMEM((CHUNK, H), jnp.float32),
            pltpu.VMEM_SHARED((N_PAD, H), jnp.float32),
        ],
    )
    def sc_kernel(xh_hbm, src_hbm, dst_hbm, zeros_hbm, out_hbm,
                  src_v, dst_v, rows_v, acc_sh):
        c = lax.axis_index("c")
        s = lax.axis_index("s")
        pltpu.sync_copy(zeros_hbm, rows_v)
        for z in range(ROWS_PER_TILE // CHUNK):
            pltpu.sync_copy(
                rows_v, acc_sh.at[pl.ds(s * ROWS_PER_TILE + z * CHUNK, CHUNK)])
        plsc.subcore_barrier()

        @pl.loop(0, MAX_CHUNKS_PER_TILE)
        def _(k):
            i = k * N_TILES + s

            @pl.when(i < N_CHUNKS)
            def _():
                pltpu.sync_copy(src_hbm.at[c, i], src_v)
                pltpu.sync_copy(dst_hbm.at[i], dst_v)
                pltpu.sync_copy(xh_hbm.at[src_v], rows_v)
                pltpu.sync_copy(rows_v, acc_sh.at[dst_v], add=True)

        plsc.subcore_barrier()
        pltpu.sync_copy(
            acc_sh.at[pl.ds(s * ROWS_PER_TILE, ROWS_PER_TILE)],
            out_hbm.at[pl.ds(c * N_PAD + s * ROWS_PER_TILE, ROWS_PER_TILE)],
        )

    return sc_kernel(xh, src_idx, dst_idx, zeros)


def _tc_root_body(x_ref, wr_ref, b_ref, o_ref):
    o_ref[...] = jnp.dot(x_ref[...], wr_ref[...],
                         preferred_element_type=jnp.float32) + b_ref[...]


def _tc_root(x, wr, b2d):
    n = x.shape[0]
    blk = 1000
    return pl.pallas_call(
        _tc_root_body,
        grid=(n // blk,),
        in_specs=[
            pl.BlockSpec((blk, D), lambda i: (i, 0)),
            pl.BlockSpec((D, D), lambda i: (0, 0)),
            pl.BlockSpec((1, D), lambda i: (0, 0)),
        ],
        out_specs=pl.BlockSpec((blk, D), lambda i: (i, 0)),
        out_shape=jax.ShapeDtypeStruct((n, D), jnp.float32),
    )(x, wr, b2d)


def _tc_finish_body(a0_ref, a1_ref, z_ref, w0_ref, w1_ref, o_ref):
    y = jnp.dot(a0_ref[...], w0_ref[...], preferred_element_type=jnp.float32)
    y = y + jnp.dot(a1_ref[...], w1_ref[...], preferred_element_type=jnp.float32)
    y = y + z_ref[...]
    m = jnp.max(y, axis=-1, keepdims=True)
    t = y - m
    lse = jnp.log(jnp.sum(jnp.exp(t), axis=-1, keepdims=True))
    o_ref[...] = t - lse


def _tc_finish(agg0, agg1, z, w0, w1):
    n = z.shape[0]
    blk = 1000
    return pl.pallas_call(
        _tc_finish_body,
        grid=(n // blk,),
        in_specs=[
            pl.BlockSpec((blk, H), lambda i: (i, 0)),
            pl.BlockSpec((blk, H), lambda i: (i, 0)),
            pl.BlockSpec((blk, D), lambda i: (i, 0)),
            pl.BlockSpec((H, D), lambda i: (0, 0)),
            pl.BlockSpec((H, D), lambda i: (0, 0)),
        ],
        out_specs=pl.BlockSpec((blk, D), lambda i: (i, 0)),
        out_shape=jax.ShapeDtypeStruct((n, D), jnp.float32),
    )(agg0, agg1, z, w0, w1)


def kernel(x, edge_index, W_rel, W_root, b):
    src = edge_index[0]
    dst = edge_index[1]
    xh = jnp.concatenate([x[:, :H], x[:, H:]], axis=0)
    srcs = src.reshape(N_CHUNKS, CHUNK)
    src_idx = jnp.stack([srcs, srcs + N_NODES])
    dst_idx = dst.reshape(N_CHUNKS, CHUNK)
    zeros = jnp.zeros((CHUNK, H), jnp.float32)

    agg_cat = _sc_segment_sum(xh, src_idx, dst_idx, zeros)
    z = _tc_root(x, W_root.T, b.reshape(1, D))

    out = _tc_finish(
        agg_cat[:N_NODES], agg_cat[N_PAD:N_PAD + N_NODES], z,
        W_rel[:, :H].T, W_rel[:, H:].T,
    )
    return out
